# Initial kernel scaffold; baseline (speedup 1.0000x reference)
import jax
import jax.numpy as jnp
from jax.experimental import pallas as pl


def kernel(self_vecs, neigh_vecs, W_neigh, b_neigh, W_self, b_self):
    raise NotImplementedError("write your pallas kernel here")


# speedup vs baseline: 1.4510x; 1.4510x over previous
import jax
import jax.numpy as jnp
from jax.experimental import pallas as pl

_D = 128
_BLK = 200


def _body(s_ref, n_ref, w_ref, b_ref, o_ref):
    ssum = jnp.sum(s_ref[...], axis=1)
    nsum = jnp.sum(n_ref[...], axis=1)
    x = jnp.concatenate([ssum, nsum], axis=-1)
    y = jnp.dot(x, w_ref[...], preferred_element_type=jnp.float32)
    o_ref[...] = jnp.maximum(y + b_ref[...], 0.0)


def kernel(self_vecs, neigh_vecs, W_neigh, b_neigh, W_self, b_self):
    n_nodes, s_self, d = self_vecs.shape
    s_neigh = neigh_vecs.shape[1]
    w = jnp.concatenate([W_self / s_self, W_neigh / s_neigh], axis=0)
    b = (b_self + b_neigh).reshape(1, d)

    blk = _BLK
    grid = (n_nodes // blk,)

    return pl.pallas_call(
        _body,
        grid=grid,
        in_specs=[
            pl.BlockSpec((blk, s_self, d), lambda i: (i, 0, 0)),
            pl.BlockSpec((blk, s_neigh, d), lambda i: (i, 0, 0)),
            pl.BlockSpec((2 * d, d), lambda i: (0, 0)),
            pl.BlockSpec((1, d), lambda i: (0, 0)),
        ],
        out_specs=pl.BlockSpec((blk, d), lambda i: (i, 0)),
        out_shape=jax.ShapeDtypeStruct((n_nodes, d), jnp.float32),
    )(self_vecs, neigh_vecs, w, b)


# speedup vs baseline: 1.7747x; 1.2231x over previous
import jax
import jax.numpy as jnp
from jax.experimental import pallas as pl

_D = 128
_BLK = 400


def _body(s_ref, n_ref, w_ref, b_ref, o_ref):
    ssum = jnp.sum(s_ref[...], axis=1)
    nsum = jnp.sum(n_ref[...], axis=1)
    x = jnp.concatenate([ssum, nsum], axis=-1)
    y = jnp.dot(x, w_ref[...], preferred_element_type=jnp.float32)
    o_ref[...] = jnp.maximum(y + b_ref[...], 0.0)


def kernel(self_vecs, neigh_vecs, W_neigh, b_neigh, W_self, b_self):
    n_nodes, s_self, d = self_vecs.shape
    s_neigh = neigh_vecs.shape[1]
    w = jnp.concatenate([W_self / s_self, W_neigh / s_neigh], axis=0)
    b = (b_self + b_neigh).reshape(1, d)

    blk = _BLK
    grid = (n_nodes // blk,)

    return pl.pallas_call(
        _body,
        grid=grid,
        in_specs=[
            pl.BlockSpec((blk, s_self, d), lambda i: (i, 0, 0)),
            pl.BlockSpec((blk, s_neigh, d), lambda i: (i, 0, 0)),
            pl.BlockSpec((2 * d, d), lambda i: (0, 0)),
            pl.BlockSpec((1, d), lambda i: (0, 0)),
        ],
        out_specs=pl.BlockSpec((blk, d), lambda i: (i, 0)),
        out_shape=jax.ShapeDtypeStruct((n_nodes, d), jnp.float32),
    )(self_vecs, neigh_vecs, w, b)
